# baseline (speedup 1.0000x reference)
import functools
import math

import jax
import jax.numpy as jnp
from jax import lax
from jax.experimental import pallas as pl
from jax.experimental.pallas import tpu as pltpu
from jax.experimental.pallas import tpu_sc as plsc

NC = 2
NS = 16
L = 16

C = 256
SS = 4
SR = C // SS


def _rsqrt(a):
    i = lax.bitcast_convert_type(a, jnp.int32)
    i = 0x5F3759DF - lax.shift_right_logical(i, 1)
    y = lax.bitcast_convert_type(i, jnp.float32)
    for _ in range(3):
        y = y * (1.5 - 0.5 * a * y * y)
    return y


def _make_sc_kernel(N, V, P, D):
    NW = NC * NS
    per_w = N // NW
    nch = per_w // C
    groups = C // L
    scale = math.sqrt(D)
    mesh = plsc.VectorSubcoreMesh(core_axis_name="c", subcore_axis_name="s")

    @functools.partial(
        pl.kernel,
        mesh=mesh,
        compiler_params=pltpu.CompilerParams(
            needs_layout_passes=False, use_tc_tiling_on_sc=False),
        out_type=jax.ShapeDtypeStruct((N, D), jnp.float32),
        scratch_types=[
            pltpu.VMEM((C,), jnp.int32),
            pltpu.VMEM((C,), jnp.int32),
            pltpu.VMEM((C,), jnp.int32),
            pltpu.VMEM((C,), jnp.int32),
            pltpu.VMEM((C, D), jnp.float32),
            pltpu.VMEM((C, D), jnp.float32),
            pltpu.VMEM((C, D), jnp.float32),
            pltpu.VMEM((C, D), jnp.float32),
            pltpu.VMEM((P * D,), jnp.float32),
            pltpu.VMEM((D,), jnp.float32),
            pltpu.VMEM((D,), jnp.float32),
            pltpu.VMEM((D, L), jnp.float32),
            pltpu.SemaphoreType.DMA,
            pltpu.SemaphoreType.DMA,
            pltpu.SemaphoreType.DMA,
            pltpu.SemaphoreType.DMA,
        ],
    )
    def sc_kernel(idx_hbm, pid_hbm, item_hbm, pos_hbm, gam_hbm, bet_hbm,
                  out_hbm, idx0, idx1, pid0, pid1, rows0, rows1, ob0, ob1,
                  pos_v, gam_v, bet_v, xT, sg0, sg1, so0, so1):
        idxs, pids, rows, obs = [idx0, idx1], [pid0, pid1], [rows0, rows1], [ob0, ob1]
        sgs, sos = [sg0, sg1], [so0, so1]
        wid = lax.axis_index("s") * NC + lax.axis_index("c")
        base = wid * per_w

        pltpu.sync_copy(pos_hbm, pos_v)
        pltpu.sync_copy(gam_hbm, gam_v)
        pltpu.sync_copy(bet_hbm, bet_v)

        lanes = lax.broadcasted_iota(jnp.int32, (L,), 0)
        gvec = [gam_v[pl.ds(k * L, L)] for k in range(D // L)]
        bvec = [bet_v[pl.ds(k * L, L)] for k in range(D // L)]

        def fire_in(j, s):
            row0 = base + j * C
            pltpu.sync_copy(idx_hbm.at[pl.ds(row0, C)], idxs[s])
            pltpu.sync_copy(pid_hbm.at[pl.ds(row0, C)], pids[s])
            for k in range(SS):
                pltpu.async_copy(
                    item_hbm.at[idxs[s].at[pl.ds(k * SR, SR)]],
                    rows[s].at[pl.ds(k * SR, SR)],
                    sgs[s])

        def wait_in(s):
            for k in range(SS):
                pltpu.make_async_copy(
                    item_hbm.at[idxs[s].at[pl.ds(0, SR)]],
                    rows[s].at[pl.ds(0, SR)],
                    sgs[s]).wait()

        def compute(s):
            rv, pv_ids, ob = rows[s], pids[s], obs[s]

            def group_body(gi, _):
                g0 = gi * L
                ridx = g0 + lanes
                pid16 = pv_ids[pl.ds(g0, L)]
                pbase = pid16 * D

                def col1(j, carry):
                    sacc, qacc = carry
                    cj = jnp.full((L,), j, jnp.int32)
                    iv = plsc.load_gather(rv, [ridx, cj])
                    pvv = plsc.load_gather(pos_v, [pbase + j])
                    x = iv * scale + pvv
                    xT[j] = x
                    return sacc + x, qacc + x * x

                zero = jnp.zeros((L,), jnp.float32)
                sacc, qacc = lax.fori_loop(0, D, col1, (zero, zero))
                mu = sacc * (1.0 / D)
                var = qacc * (1.0 / D) - mu * mu
                rstd = _rsqrt(var + 1e-5)

                def col2(j, _):
                    cj = jnp.full((L,), j, jnp.int32)
                    xh = (xT[j] - mu) * rstd
                    plsc.store_scatter(ob, [ridx, cj], xh)
                    return 0

                lax.fori_loop(0, D, col2, 0)

                def row3(r, _):
                    rr = g0 + r
                    for k in range(D // L):
                        v = ob[rr, pl.ds(k * L, L)]
                        ob[rr, pl.ds(k * L, L)] = v * gvec[k] + bvec[k]
                    return 0

                lax.fori_loop(0, L, row3, 0)
                return 0

            lax.fori_loop(0, groups, group_body, 0)

        fire_in(0, 0)

        def pair_body(ci, _):
            for b in range(2):
                j = 2 * ci + b

                @pl.when(j + 1 < nch)
                def _():
                    fire_in(j + 1, 1 - b)

                wait_in(b)

                @pl.when(j >= 2)
                def _():
                    pltpu.make_async_copy(
                        obs[b], out_hbm.at[pl.ds(base, C)], sos[b]).wait()

                compute(b)
                pltpu.async_copy(
                    obs[b], out_hbm.at[pl.ds(base + j * C, C)], sos[b])
            return 0

        lax.fori_loop(0, nch // 2, pair_body, 0)
        pltpu.make_async_copy(obs[0], out_hbm.at[pl.ds(base, C)], sos[0]).wait()
        pltpu.make_async_copy(obs[1], out_hbm.at[pl.ds(base, C)], sos[1]).wait()

    return sc_kernel


def kernel(input_sequence, position_ids, item_table, pos_table, ln_gamma, ln_beta):
    I, B = input_sequence.shape
    V, D = item_table.shape
    P = pos_table.shape[0]
    N = I * B
    sc = _make_sc_kernel(N, V, P, D)
    out = sc(
        input_sequence.reshape(N),
        position_ids.reshape(N),
        item_table,
        pos_table.reshape(P * D),
        ln_gamma,
        ln_beta,
    )
    return out.reshape(I, B, D)


# speedup vs baseline: 1.4703x; 1.4703x over previous
import functools
import math

import jax
import jax.numpy as jnp
from jax import lax
from jax.experimental import pallas as pl
from jax.experimental.pallas import tpu as pltpu
from jax.experimental.pallas import tpu_sc as plsc

NC = 2
NS = 16
L = 16

C = 256
SS = 4
SR = C // SS


def _rsqrt(a):
    i = lax.bitcast_convert_type(a, jnp.int32)
    i = 0x5F3759DF - lax.shift_right_logical(i, 1)
    y = lax.bitcast_convert_type(i, jnp.float32)
    for _ in range(2):
        y = y * (1.5 - 0.5 * a * y * y)
    return y


def _make_sc_kernel(N, V, P, D):
    NW = NC * NS
    per_w = N // NW
    nch = per_w // C
    groups = C // L
    scale = math.sqrt(D)
    mesh = plsc.VectorSubcoreMesh(core_axis_name="c", subcore_axis_name="s")

    @functools.partial(
        pl.kernel,
        mesh=mesh,
        compiler_params=pltpu.CompilerParams(
            needs_layout_passes=False, use_tc_tiling_on_sc=False),
        out_type=jax.ShapeDtypeStruct((N * D,), jnp.float32),
        scratch_types=[
            pltpu.VMEM((C,), jnp.int32),
            pltpu.VMEM((C,), jnp.int32),
            pltpu.VMEM((C,), jnp.int32),
            pltpu.VMEM((C,), jnp.int32),
            pltpu.VMEM((C, D), jnp.float32),
            pltpu.VMEM((C, D), jnp.float32),
            pltpu.VMEM((C * D,), jnp.float32),
            pltpu.VMEM((C * D,), jnp.float32),
            pltpu.VMEM((P * D,), jnp.float32),
            pltpu.VMEM((D,), jnp.float32),
            pltpu.VMEM((D,), jnp.float32),
            pltpu.VMEM((D, L), jnp.float32),
            pltpu.SemaphoreType.DMA,
            pltpu.SemaphoreType.DMA,
            pltpu.SemaphoreType.DMA,
            pltpu.SemaphoreType.DMA,
        ],
    )
    def sc_kernel(idx_hbm, pid_hbm, item_hbm, pos_hbm, gam_hbm, bet_hbm,
                  out_hbm, idx0, idx1, pid0, pid1, rows0, rows1, ob0, ob1,
                  pos_v, gam_v, bet_v, xT, sg0, sg1, so0, so1):
        idxs, pids, rows, obs = [idx0, idx1], [pid0, pid1], [rows0, rows1], [ob0, ob1]
        sgs, sos = [sg0, sg1], [so0, so1]
        wid = lax.axis_index("s") * NC + lax.axis_index("c")
        base = wid * per_w

        pltpu.sync_copy(pos_hbm, pos_v)
        pltpu.sync_copy(gam_hbm, gam_v)
        pltpu.sync_copy(bet_hbm, bet_v)

        lanes = lax.broadcasted_iota(jnp.int32, (L,), 0)
        gvec = [gam_v[pl.ds(k * L, L)] for k in range(D // L)]
        bvec = [bet_v[pl.ds(k * L, L)] for k in range(D // L)]
        zero = jnp.zeros((L,), jnp.float32)

        def fire_in(j, s):
            row0 = base + j * C
            pltpu.sync_copy(idx_hbm.at[pl.ds(row0, C)], idxs[s])
            pltpu.sync_copy(pid_hbm.at[pl.ds(row0, C)], pids[s])
            for k in range(SS):
                pltpu.async_copy(
                    item_hbm.at[idxs[s].at[pl.ds(k * SR, SR)]],
                    rows[s].at[pl.ds(k * SR, SR)],
                    sgs[s])

        def wait_in(s):
            for k in range(SS):
                pltpu.make_async_copy(
                    item_hbm.at[idxs[s].at[pl.ds(0, SR)]],
                    rows[s].at[pl.ds(0, SR)],
                    sgs[s]).wait()

        def compute(s):
            rv, pv_ids, ob = rows[s], pids[s], obs[s]

            def group_body(gi, _):
                g0 = gi * L
                ridx = g0 + lanes
                obase = g0 * D + lanes * D
                pid16 = pv_ids[pl.ds(g0, L)]
                pbase = pid16 * D

                @plsc.parallel_loop(0, D, step=2, unroll=4,
                                    carry=(zero, zero, zero, zero))
                def col1(j, carry):
                    s0, q0, s1, q1 = carry
                    ca = jnp.full((L,), j, jnp.int32)
                    iva = plsc.load_gather(rv, [ridx, ca])
                    pva = plsc.load_gather(pos_v, [pbase + j])
                    xa = iva * scale + pva
                    xT[j] = xa
                    cb = ca + 1
                    ivb = plsc.load_gather(rv, [ridx, cb])
                    pvb = plsc.load_gather(pos_v, [pbase + j + 1])
                    xb = ivb * scale + pvb
                    xT[j + 1] = xb
                    return s0 + xa, q0 + xa * xa, s1 + xb, q1 + xb * xb

                s0, q0, s1, q1 = col1
                mu = (s0 + s1) * (1.0 / D)
                var = (q0 + q1) * (1.0 / D) - mu * mu
                rstd = _rsqrt(var + 1e-5)
                nmu = mu * rstd

                @plsc.parallel_loop(0, D, step=2, unroll=4)
                def col2(j):
                    xha = xT[j] * rstd - nmu
                    plsc.store_scatter(ob, [obase + j], xha)
                    xhb = xT[j + 1] * rstd - nmu
                    plsc.store_scatter(ob, [obase + j + 1], xhb)

                return 0

            lax.fori_loop(0, groups, group_body, 0)

            @plsc.parallel_loop(0, C * D, step=4 * L, unroll=4)
            def affine(i):
                for t in range(D // L):
                    v = ob[pl.ds(i + t * L, L)]
                    ob[pl.ds(i + t * L, L)] = v * gvec[t] + bvec[t]

        fire_in(0, 0)

        def pair_body(ci, _):
            for b in range(2):
                j = 2 * ci + b

                @pl.when(j + 1 < nch)
                def _():
                    fire_in(j + 1, 1 - b)

                wait_in(b)

                @pl.when(j >= 2)
                def _():
                    pltpu.make_async_copy(
                        obs[b], out_hbm.at[pl.ds(base * D, C * D)],
                        sos[b]).wait()

                compute(b)
                pltpu.async_copy(
                    obs[b], out_hbm.at[pl.ds((base + j * C) * D, C * D)],
                    sos[b])
            return 0

        lax.fori_loop(0, nch // 2, pair_body, 0)
        pltpu.make_async_copy(
            obs[0], out_hbm.at[pl.ds(base * D, C * D)], sos[0]).wait()
        pltpu.make_async_copy(
            obs[1], out_hbm.at[pl.ds(base * D, C * D)], sos[1]).wait()

    return sc_kernel


def kernel(input_sequence, position_ids, item_table, pos_table, ln_gamma, ln_beta):
    I, B = input_sequence.shape
    V, D = item_table.shape
    P = pos_table.shape[0]
    N = I * B
    sc = _make_sc_kernel(N, V, P, D)
    out = sc(
        input_sequence.reshape(N),
        position_ids.reshape(N),
        item_table,
        pos_table.reshape(P * D),
        ln_gamma,
        ln_beta,
    )
    return out.reshape(I, B, D)


# speedup vs baseline: 2.7169x; 1.8479x over previous
import functools
import math

import jax
import jax.numpy as jnp
from jax import lax
from jax.experimental import pallas as pl
from jax.experimental.pallas import tpu as pltpu
from jax.experimental.pallas import tpu_sc as plsc

NC = 2
NS = 16
L = 16

C = 256
SS = 4
SR = C // SS


def _rsqrt(a):
    i = lax.bitcast_convert_type(a, jnp.int32)
    i = 0x5F3759DF - lax.shift_right_logical(i, 1)
    y = lax.bitcast_convert_type(i, jnp.float32)
    for _ in range(2):
        y = y * (1.5 - 0.5 * a * y * y)
    return y


def _make_sc_kernel(N, V, P, D):
    NW = NC * NS
    per_w = N // NW
    nch = per_w // C
    groups = C // L
    scale = math.sqrt(D)
    IT = N // (8 * 128 * 32)
    TJ = D // 8
    mesh = plsc.VectorSubcoreMesh(core_axis_name="c", subcore_axis_name="s")

    @functools.partial(
        pl.kernel,
        mesh=mesh,
        compiler_params=pltpu.CompilerParams(
            needs_layout_passes=False, use_tc_tiling_on_sc=False),
        out_type=jax.ShapeDtypeStruct((IT * 8, TJ, 32, 8, 128), jnp.float32),
        scratch_types=[
            pltpu.VMEM((C,), jnp.int32),
            pltpu.VMEM((C,), jnp.int32),
            pltpu.VMEM((C,), jnp.int32),
            pltpu.VMEM((C,), jnp.int32),
            pltpu.VMEM((C, D), jnp.float32),
            pltpu.VMEM((C, D), jnp.float32),
            pltpu.VMEM((2, D, 128), jnp.float32),
            pltpu.VMEM((2, D, 128), jnp.float32),
            pltpu.VMEM((D,), jnp.float32),
            pltpu.VMEM((D,), jnp.float32),
            pltpu.VMEM((D, L), jnp.float32),
            pltpu.VMEM((D, L), jnp.float32),
            pltpu.VMEM((D, L), jnp.float32),
            pltpu.SemaphoreType.DMA,
            pltpu.SemaphoreType.DMA,
            pltpu.SemaphoreType.DMA,
            pltpu.SemaphoreType.DMA,
        ],
    )
    def sc_kernel(idx_hbm, pid_hbm, item_hbm, pos_hbm, gam_hbm, bet_hbm,
                  out_hbm, idx0, idx1, pid0, pid1, rows0, rows1, ob0, ob1,
                  gam_v, bet_v, xT, gspl, bspl, sg0, sg1, so0, so1):
        idxs, pids, rows, obs = [idx0, idx1], [pid0, pid1], [rows0, rows1], [ob0, ob1]
        sgs, sos = [sg0, sg1], [so0, so1]
        wid = lax.axis_index("s") * NC + lax.axis_index("c")
        base = wid * per_w
        u0 = base // 1024

        pltpu.sync_copy(gam_hbm, gam_v)
        pltpu.sync_copy(bet_hbm, bet_v)

        lanes = lax.broadcasted_iota(jnp.int32, (L,), 0)
        zero = jnp.zeros((L,), jnp.float32)

        @plsc.parallel_loop(0, D, step=1, unroll=4)
        def _build_splats(j):
            cj = jnp.full((L,), j, jnp.int32)
            gspl[j] = plsc.load_gather(gam_v, [cj])
            bspl[j] = plsc.load_gather(bet_v, [cj])

        def fire_in(j, s):
            row0 = base + j * C
            pltpu.sync_copy(idx_hbm.at[pl.ds(row0, C)], idxs[s])
            pltpu.sync_copy(pid_hbm.at[pl.ds(row0, C)], pids[s])
            for k in range(SS):
                pltpu.async_copy(
                    item_hbm.at[idxs[s].at[pl.ds(k * SR, SR)]],
                    rows[s].at[pl.ds(k * SR, SR)],
                    sgs[s])

        def add_pos(s):
            for k in range(SS):
                pltpu.async_copy(
                    pos_hbm.at[pids[s].at[pl.ds(k * SR, SR)]],
                    rows[s].at[pl.ds(k * SR, SR)],
                    sgs[s], add=True)

        def wait_in(s, n=SS):
            for k in range(n):
                pltpu.make_async_copy(
                    item_hbm.at[idxs[s].at[pl.ds(0, SR)]],
                    rows[s].at[pl.ds(0, SR)],
                    sgs[s]).wait()

        def compute(s):
            rv, pv_ids, ob = rows[s], pids[s], obs[s]

            def group_body(gi, _):
                g0 = gi * L
                il = gi // 8
                l0 = (gi % 8) * L
                ridx = g0 + lanes

                @plsc.parallel_loop(0, D, step=2, unroll=4,
                                    carry=(zero, zero, zero, zero))
                def col1(j, carry):
                    s0, q0, s1, q1 = carry
                    ca = jnp.full((L,), j, jnp.int32)
                    xa = plsc.load_gather(rv, [ridx, ca])
                    xT[j] = xa
                    xb = plsc.load_gather(rv, [ridx, ca + 1])
                    xT[j + 1] = xb
                    return s0 + xa, q0 + xa * xa, s1 + xb, q1 + xb * xb

                s0, q0, s1, q1 = col1
                mu = (s0 + s1) * (1.0 / D)
                var = (q0 + q1) * (1.0 / D) - mu * mu
                rstd = _rsqrt(var + 1e-5 / D)
                nmu = mu * rstd

                @plsc.parallel_loop(0, D, step=2, unroll=4)
                def col2(j):
                    ob[il, j, pl.ds(l0, L)] = xT[j] * rstd - nmu
                    ob[il, j + 1, pl.ds(l0, L)] = xT[j + 1] * rstd - nmu

                return 0

            lax.fori_loop(0, groups, group_body, 0)

            @plsc.parallel_loop(0, 2 * D, step=1, unroll=2)
            def affine(r):
                il = r // D
                j = r % D
                g = gspl[j]
                b = bspl[j]
                for t in range(128 // L):
                    v = ob[il, j, pl.ds(t * L, L)]
                    ob[il, j, pl.ds(t * L, L)] = v * g + b

        def fire_out(j, s):
            u = u0 + j // 4
            it = u // 32
            bt = u % 32
            is0 = (j % 4) * 2
            for il in range(2):
                i = it * 8 + is0 + il
                for tj in range(TJ):
                    pltpu.async_copy(
                        obs[s].at[il, pl.ds(tj * 8, 8), :],
                        out_hbm.at[i, tj, bt],
                        sos[s])

        def wait_out(s):
            for _ in range(2 * TJ):
                pltpu.make_async_copy(
                    obs[s].at[0, pl.ds(0, 8), :],
                    out_hbm.at[0, 0, 0],
                    sos[s]).wait()

        fire_in(0, 0)

        def pair_body(ci, _):
            for b in range(2):
                j = 2 * ci + b

                @pl.when(j + 1 < nch)
                def _():
                    fire_in(j + 1, 1 - b)

                wait_in(b)
                add_pos(b)

                @pl.when(j >= 2)
                def _():
                    wait_out(b)

                wait_in(b)

                compute(b)
                fire_out(j, b)
            return 0

        lax.fori_loop(0, nch // 2, pair_body, 0)
        wait_out(0)
        wait_out(1)

    return sc_kernel


def kernel(input_sequence, position_ids, item_table, pos_table, ln_gamma, ln_beta):
    I, B = input_sequence.shape
    V, D = item_table.shape
    P = pos_table.shape[0]
    N = I * B

    def native_flat(a):
        return (a.reshape(I // 8, 8, B // 128, 128)
                 .transpose(0, 2, 1, 3)
                 .reshape(N))

    sc = _make_sc_kernel(N, V, P, D)
    out5 = sc(
        native_flat(input_sequence),
        native_flat(position_ids),
        item_table,
        pos_table * (1.0 / math.sqrt(D)),
        ln_gamma,
        ln_beta,
    )
    return out5.transpose(0, 2, 4, 1, 3).reshape(I, B, D)


# speedup vs baseline: 2.8915x; 1.0643x over previous
import functools
import math

import jax
import jax.numpy as jnp
from jax import lax
from jax.experimental import pallas as pl
from jax.experimental.pallas import tpu as pltpu
from jax.experimental.pallas import tpu_sc as plsc

NC = 2
NS = 16
L = 16

C = 256
SS = 4
SR = C // SS


def _rsqrt(a):
    i = lax.bitcast_convert_type(a, jnp.int32)
    i = 0x5F3759DF - lax.shift_right_logical(i, 1)
    y = lax.bitcast_convert_type(i, jnp.float32)
    for _ in range(2):
        y = y * (1.5 - 0.5 * a * y * y)
    return y


def _make_sc_kernel(N, V, P, D):
    NW = NC * NS
    per_w = N // NW
    nch = per_w // C
    groups = C // L
    scale = math.sqrt(D)
    IT = N // (8 * 128 * 32)
    TJ = D // 8
    mesh = plsc.VectorSubcoreMesh(core_axis_name="c", subcore_axis_name="s")

    @functools.partial(
        pl.kernel,
        mesh=mesh,
        compiler_params=pltpu.CompilerParams(
            needs_layout_passes=False, use_tc_tiling_on_sc=False),
        out_type=jax.ShapeDtypeStruct((IT * 8, TJ, 32, 8, 128), jnp.float32),
        scratch_types=[
            pltpu.VMEM((C,), jnp.int32),
            pltpu.VMEM((C,), jnp.int32),
            pltpu.VMEM((C,), jnp.int32),
            pltpu.VMEM((C,), jnp.int32),
            pltpu.VMEM((C, D), jnp.float32),
            pltpu.VMEM((C, D), jnp.float32),
            pltpu.VMEM_SHARED((200, 64), jnp.float32),
            pltpu.VMEM((2, D, 128), jnp.float32),
            pltpu.VMEM((2, D, 128), jnp.float32),
            pltpu.VMEM((D,), jnp.float32),
            pltpu.VMEM((D,), jnp.float32),
            pltpu.VMEM((D, L), jnp.float32),
            pltpu.VMEM((D, L), jnp.float32),
            pltpu.VMEM((D, L), jnp.float32),
            pltpu.SemaphoreType.DMA,
            pltpu.SemaphoreType.DMA,
            pltpu.SemaphoreType.DMA,
            pltpu.SemaphoreType.DMA,
        ],
    )
    def sc_kernel(idx_hbm, pid_hbm, item_hbm, pos_hbm, gam_hbm, bet_hbm,
                  out_hbm, idx0, idx1, pid0, pid1, rows0, rows1, pos_v, ob0, ob1,
                  gam_v, bet_v, xT, gspl, bspl, sg0, sg1, so0, so1):
        idxs, pids, rows, obs = [idx0, idx1], [pid0, pid1], [rows0, rows1], [ob0, ob1]
        sgs, sos = [sg0, sg1], [so0, so1]
        wid = lax.axis_index("s") * NC + lax.axis_index("c")
        base = wid * per_w
        u0 = base // 1024

        @pl.when(lax.axis_index("s") == 0)
        def _():
            pltpu.sync_copy(pos_hbm, pos_v)
        plsc.subcore_barrier()
        pltpu.sync_copy(gam_hbm, gam_v)
        pltpu.sync_copy(bet_hbm, bet_v)

        lanes = lax.broadcasted_iota(jnp.int32, (L,), 0)
        zero = jnp.zeros((L,), jnp.float32)

        @plsc.parallel_loop(0, D, step=1, unroll=4)
        def _build_splats(j):
            cj = jnp.full((L,), j, jnp.int32)
            gspl[j] = plsc.load_gather(gam_v, [cj])
            bspl[j] = plsc.load_gather(bet_v, [cj])

        def fire_in(j, s):
            row0 = base + j * C
            pltpu.sync_copy(idx_hbm.at[pl.ds(row0, C)], idxs[s])
            pltpu.sync_copy(pid_hbm.at[pl.ds(row0, C)], pids[s])
            for k in range(SS):
                pltpu.async_copy(
                    item_hbm.at[idxs[s].at[pl.ds(k * SR, SR)]],
                    rows[s].at[pl.ds(k * SR, SR)],
                    sgs[s])

        def add_pos(s):
            for k in range(SS):
                pltpu.async_copy(
                    pos_v.at[pids[s].at[pl.ds(k * SR, SR)]],
                    rows[s].at[pl.ds(k * SR, SR)],
                    sgs[s], add=True)

        def wait_in(s, n=SS):
            for k in range(n):
                pltpu.make_async_copy(
                    item_hbm.at[idxs[s].at[pl.ds(0, SR)]],
                    rows[s].at[pl.ds(0, SR)],
                    sgs[s]).wait()

        def compute(s):
            rv, pv_ids, ob = rows[s], pids[s], obs[s]

            def group_body(gi, _):
                g0 = gi * L
                il = gi // 8
                l0 = (gi % 8) * L
                ridx = g0 + lanes

                @plsc.parallel_loop(0, D, step=2, unroll=4,
                                    carry=(zero, zero, zero, zero))
                def col1(j, carry):
                    s0, q0, s1, q1 = carry
                    ca = jnp.full((L,), j, jnp.int32)
                    xa = plsc.load_gather(rv, [ridx, ca])
                    xT[j] = xa
                    xb = plsc.load_gather(rv, [ridx, ca + 1])
                    xT[j + 1] = xb
                    return s0 + xa, q0 + xa * xa, s1 + xb, q1 + xb * xb

                s0, q0, s1, q1 = col1
                mu = (s0 + s1) * (1.0 / D)
                var = (q0 + q1) * (1.0 / D) - mu * mu
                rstd = _rsqrt(var + 1e-5 / D)
                nmu = mu * rstd

                @plsc.parallel_loop(0, D, step=2, unroll=4)
                def col2(j):
                    ob[il, j, pl.ds(l0, L)] = xT[j] * rstd - nmu
                    ob[il, j + 1, pl.ds(l0, L)] = xT[j + 1] * rstd - nmu

                return 0

            lax.fori_loop(0, groups, group_body, 0)

            @plsc.parallel_loop(0, 2 * D, step=1, unroll=2)
            def affine(r):
                il = r // D
                j = r % D
                g = gspl[j]
                b = bspl[j]
                for t in range(128 // L):
                    v = ob[il, j, pl.ds(t * L, L)]
                    ob[il, j, pl.ds(t * L, L)] = v * g + b

        def fire_out(j, s):
            u = u0 + j // 4
            it = u // 32
            bt = u % 32
            is0 = (j % 4) * 2
            for il in range(2):
                i = it * 8 + is0 + il
                for tj in range(TJ):
                    pltpu.async_copy(
                        obs[s].at[il, pl.ds(tj * 8, 8), :],
                        out_hbm.at[i, tj, bt],
                        sos[s])

        def wait_out(s):
            for _ in range(2 * TJ):
                pltpu.make_async_copy(
                    obs[s].at[0, pl.ds(0, 8), :],
                    out_hbm.at[0, 0, 0],
                    sos[s]).wait()

        fire_in(0, 0)

        def pair_body(ci, _):
            for b in range(2):
                j = 2 * ci + b

                @pl.when(j + 1 < nch)
                def _():
                    fire_in(j + 1, 1 - b)

                wait_in(b)
                add_pos(b)

                @pl.when(j >= 2)
                def _():
                    wait_out(b)

                wait_in(b)

                compute(b)
                fire_out(j, b)
            return 0

        lax.fori_loop(0, nch // 2, pair_body, 0)
        wait_out(0)
        wait_out(1)

    return sc_kernel


def kernel(input_sequence, position_ids, item_table, pos_table, ln_gamma, ln_beta):
    I, B = input_sequence.shape
    V, D = item_table.shape
    P = pos_table.shape[0]
    N = I * B

    def native_flat(a):
        return (a.reshape(I // 8, 8, B // 128, 128)
                 .transpose(0, 2, 1, 3)
                 .reshape(N))

    sc = _make_sc_kernel(N, V, P, D)
    out5 = sc(
        native_flat(input_sequence),
        native_flat(position_ids),
        item_table,
        pos_table * (1.0 / math.sqrt(D)),
        ln_gamma,
        ln_beta,
    )
    return out5.transpose(0, 2, 4, 1, 3).reshape(I, B, D)


# speedup vs baseline: 3.0144x; 1.0425x over previous
import functools
import math

import jax
import jax.numpy as jnp
from jax import lax
from jax.experimental import pallas as pl
from jax.experimental.pallas import tpu as pltpu
from jax.experimental.pallas import tpu_sc as plsc

NC = 2
NS = 16
L = 16

C = 256
SS = 4
SR = C // SS


def _rsqrt(a):
    i = lax.bitcast_convert_type(a, jnp.int32)
    i = 0x5F3759DF - lax.shift_right_logical(i, 1)
    y = lax.bitcast_convert_type(i, jnp.float32)
    for _ in range(2):
        y = y * (1.5 - 0.5 * a * y * y)
    return y


def _make_sc_kernel(N, V, P, D):
    NW = NC * NS
    per_w = N // NW
    nch = per_w // C
    groups = C // L
    scale = math.sqrt(D)
    IT = N // (8 * 128 * 32)
    TJ = D // 8
    mesh = plsc.VectorSubcoreMesh(core_axis_name="c", subcore_axis_name="s")

    @functools.partial(
        pl.kernel,
        mesh=mesh,
        compiler_params=pltpu.CompilerParams(
            needs_layout_passes=False, use_tc_tiling_on_sc=False),
        out_type=jax.ShapeDtypeStruct((IT * 8, TJ, 32, 8, 128), jnp.float32),
        scratch_types=[
            pltpu.VMEM((C,), jnp.int32),
            pltpu.VMEM((C,), jnp.int32),
            pltpu.VMEM((C,), jnp.int32),
            pltpu.VMEM((C,), jnp.int32),
            pltpu.VMEM((C,), jnp.int32),
            pltpu.VMEM((C,), jnp.int32),
            pltpu.VMEM((C, D), jnp.float32),
            pltpu.VMEM((C, D), jnp.float32),
            pltpu.VMEM((C, D), jnp.float32),
            pltpu.VMEM_SHARED((200, 64), jnp.float32),
            pltpu.VMEM((2, D, 128), jnp.float32),
            pltpu.VMEM((2, D, 128), jnp.float32),
            pltpu.VMEM((D,), jnp.float32),
            pltpu.VMEM((D,), jnp.float32),
            pltpu.VMEM((D, L), jnp.float32),
            pltpu.VMEM((D, L), jnp.float32),
            pltpu.VMEM((D, L), jnp.float32),
            pltpu.SemaphoreType.DMA,
            pltpu.SemaphoreType.DMA,
            pltpu.SemaphoreType.DMA,
            pltpu.SemaphoreType.DMA,
            pltpu.SemaphoreType.DMA,
        ],
    )
    def sc_kernel(idx_hbm, pid_hbm, item_hbm, pos_hbm, gam_hbm, bet_hbm,
                  out_hbm, idx0, idx1, idx2, pid0, pid1, pid2, rows0, rows1, rows2,
                  pos_v, ob0, ob1, gam_v, bet_v, xT, gspl, bspl,
                  sg0, sg1, sg2, so0, so1):
        idxs, pids, rows = [idx0, idx1, idx2], [pid0, pid1, pid2], [rows0, rows1, rows2]
        obs = [ob0, ob1]
        sgs, sos = [sg0, sg1, sg2], [so0, so1]
        wid = lax.axis_index("s") * NC + lax.axis_index("c")
        base = wid * per_w
        u0 = base // 1024

        @pl.when(lax.axis_index("s") == 0)
        def _():
            pltpu.sync_copy(pos_hbm, pos_v)
        plsc.subcore_barrier()
        pltpu.sync_copy(gam_hbm, gam_v)
        pltpu.sync_copy(bet_hbm, bet_v)

        lanes = lax.broadcasted_iota(jnp.int32, (L,), 0)
        zero = jnp.zeros((L,), jnp.float32)

        @plsc.parallel_loop(0, D, step=1, unroll=4)
        def _build_splats(j):
            cj = jnp.full((L,), j, jnp.int32)
            gspl[j] = plsc.load_gather(gam_v, [cj])
            bspl[j] = plsc.load_gather(bet_v, [cj])

        def fire_in(j, s):
            row0 = base + j * C
            pltpu.sync_copy(idx_hbm.at[pl.ds(row0, C)], idxs[s])
            pltpu.sync_copy(pid_hbm.at[pl.ds(row0, C)], pids[s])
            for k in range(SS):
                pltpu.async_copy(
                    item_hbm.at[idxs[s].at[pl.ds(k * SR, SR)]],
                    rows[s].at[pl.ds(k * SR, SR)],
                    sgs[s])

        def add_pos(s):
            for k in range(SS):
                pltpu.async_copy(
                    pos_v.at[pids[s].at[pl.ds(k * SR, SR)]],
                    rows[s].at[pl.ds(k * SR, SR)],
                    sgs[s], add=True)

        def wait_in(s, n=SS):
            for k in range(n):
                pltpu.make_async_copy(
                    item_hbm.at[idxs[s].at[pl.ds(0, SR)]],
                    rows[s].at[pl.ds(0, SR)],
                    sgs[s]).wait()

        def compute(s, o):
            rv, ob = rows[s], obs[o]

            def group_body(gi, _):
                g0 = gi * L
                il = gi // 8
                l0 = (gi % 8) * L
                ridx = g0 + lanes

                @plsc.parallel_loop(0, D, step=2, unroll=4,
                                    carry=(zero, zero, zero, zero))
                def col1(j, carry):
                    s0, q0, s1, q1 = carry
                    ca = jnp.full((L,), j, jnp.int32)
                    xa = plsc.load_gather(rv, [ridx, ca])
                    xT[j] = xa
                    xb = plsc.load_gather(rv, [ridx, ca + 1])
                    xT[j + 1] = xb
                    return s0 + xa, q0 + xa * xa, s1 + xb, q1 + xb * xb

                s0, q0, s1, q1 = col1
                mu = (s0 + s1) * (1.0 / D)
                var = (q0 + q1) * (1.0 / D) - mu * mu
                rstd = _rsqrt(var + 1e-5 / D)
                nmu = mu * rstd

                @plsc.parallel_loop(0, D, step=2, unroll=4)
                def col2(j):
                    ob[il, j, pl.ds(l0, L)] = xT[j] * rstd - nmu
                    ob[il, j + 1, pl.ds(l0, L)] = xT[j + 1] * rstd - nmu

                return 0

            lax.fori_loop(0, groups, group_body, 0)

            @plsc.parallel_loop(0, 2 * D, step=1, unroll=2)
            def affine(r):
                il = r // D
                j = r % D
                g = gspl[j]
                b = bspl[j]
                for t in range(128 // L):
                    v = ob[il, j, pl.ds(t * L, L)]
                    ob[il, j, pl.ds(t * L, L)] = v * g + b

        def fire_out(j, s):
            u = u0 + j // 4
            it = u // 32
            bt = u % 32
            is0 = (j % 4) * 2
            for il in range(2):
                i = it * 8 + is0 + il
                for tj in range(TJ):
                    pltpu.async_copy(
                        obs[s].at[il, pl.ds(tj * 8, 8), :],
                        out_hbm.at[i, tj, bt],
                        sos[s])

        def wait_out(s):
            for _ in range(2 * TJ):
                pltpu.make_async_copy(
                    obs[s].at[0, pl.ds(0, 8), :],
                    out_hbm.at[0, 0, 0],
                    sos[s]).wait()

        fire_in(0, 0)
        fire_in(1, 1)
        wait_in(0)
        add_pos(0)

        def hex_body(ci, _):
            for b in range(6):
                j = 6 * ci + b
                sj = b % 3
                s1 = (b + 1) % 3
                s2 = (b + 2) % 3
                o = b % 2

                @pl.when(j + 2 < nch)
                def _():
                    fire_in(j + 2, s2)

                @pl.when(j + 1 < nch)
                def _():
                    wait_in(s1)
                    add_pos(s1)

                wait_in(sj)

                @pl.when(j >= 2)
                def _():
                    wait_out(o)

                compute(sj, o)
                fire_out(j, o)
            return 0

        lax.fori_loop(0, nch // 6, hex_body, 0)
        for j in range(nch - nch % 6, nch):
            sj = j % 3
            s1 = (j + 1) % 3
            s2 = (j + 2) % 3
            o = j % 2
            if j + 2 < nch:
                fire_in(j + 2, s2)
            if j + 1 < nch:
                wait_in(s1)
                add_pos(s1)
            wait_in(sj)
            if j >= 2:
                wait_out(o)
            compute(sj, o)
            fire_out(j, o)
        wait_out(0)
        wait_out(1)

    return sc_kernel


def kernel(input_sequence, position_ids, item_table, pos_table, ln_gamma, ln_beta):
    I, B = input_sequence.shape
    V, D = item_table.shape
    P = pos_table.shape[0]
    N = I * B

    def native_flat(a):
        return (a.reshape(I // 8, 8, B // 128, 128)
                 .transpose(0, 2, 1, 3)
                 .reshape(N))

    sc = _make_sc_kernel(N, V, P, D)
    out5 = sc(
        native_flat(input_sequence),
        native_flat(position_ids),
        item_table,
        pos_table * (1.0 / math.sqrt(D)),
        ln_gamma,
        ln_beta,
    )
    return out5.transpose(0, 2, 4, 1, 3).reshape(I, B, D)


# speedup vs baseline: 3.1040x; 1.0297x over previous
import functools
import math

import jax
import jax.numpy as jnp
from jax import lax
from jax.experimental import pallas as pl
from jax.experimental.pallas import tpu as pltpu
from jax.experimental.pallas import tpu_sc as plsc

NC = 2
NS = 16
L = 16

C = 256
SS = 4
SR = C // SS


def _rsqrt(a):
    i = lax.bitcast_convert_type(a, jnp.int32)
    i = 0x5F3759DF - lax.shift_right_logical(i, 1)
    y = lax.bitcast_convert_type(i, jnp.float32)
    for _ in range(2):
        y = y * (1.5 - 0.5 * a * y * y)
    return y


def _make_sc_kernel(N, V, P, D):
    NW = NC * NS
    per_w = N // NW
    nch = per_w // C
    groups = C // L
    scale = math.sqrt(D)
    IT = N // (8 * 128 * 32)
    TJ = D // 8
    mesh = plsc.VectorSubcoreMesh(core_axis_name="c", subcore_axis_name="s")

    @functools.partial(
        pl.kernel,
        mesh=mesh,
        compiler_params=pltpu.CompilerParams(
            needs_layout_passes=False, use_tc_tiling_on_sc=False),
        out_type=jax.ShapeDtypeStruct((IT * 8, TJ, 32, 8, 128), jnp.float32),
        scratch_types=[
            pltpu.VMEM((C,), jnp.int32),
            pltpu.VMEM((C,), jnp.int32),
            pltpu.VMEM((C,), jnp.int32),
            pltpu.VMEM((C,), jnp.int32),
            pltpu.VMEM((C,), jnp.int32),
            pltpu.VMEM((C,), jnp.int32),
            pltpu.VMEM((C, D), jnp.float32),
            pltpu.VMEM((C, D), jnp.float32),
            pltpu.VMEM((C, D), jnp.float32),
            pltpu.VMEM_SHARED((200, 64), jnp.float32),
            pltpu.VMEM((2, D, 128), jnp.float32),
            pltpu.VMEM((2, D, 128), jnp.float32),
            pltpu.VMEM((D,), jnp.float32),
            pltpu.VMEM((D,), jnp.float32),
            pltpu.VMEM((D, L), jnp.float32),
            pltpu.VMEM((D, L), jnp.float32),
            pltpu.VMEM((D, L), jnp.float32),
            pltpu.SemaphoreType.DMA,
            pltpu.SemaphoreType.DMA,
            pltpu.SemaphoreType.DMA,
            pltpu.SemaphoreType.DMA,
            pltpu.SemaphoreType.DMA,
        ],
    )
    def sc_kernel(idx_hbm, pid_hbm, item_hbm, pos_hbm, gam_hbm, bet_hbm,
                  out_hbm, idx0, idx1, idx2, pid0, pid1, pid2, rows0, rows1, rows2,
                  pos_v, ob0, ob1, gam_v, bet_v, xT, gspl, bspl,
                  sg0, sg1, sg2, so0, so1):
        idxs, pids, rows = [idx0, idx1, idx2], [pid0, pid1, pid2], [rows0, rows1, rows2]
        obs = [ob0, ob1]
        sgs, sos = [sg0, sg1, sg2], [so0, so1]
        wid = lax.axis_index("s") * NC + lax.axis_index("c")
        base = wid * per_w
        u0 = base // 1024

        @pl.when(lax.axis_index("s") == 0)
        def _():
            pltpu.sync_copy(pos_hbm, pos_v)
        plsc.subcore_barrier()
        pltpu.sync_copy(gam_hbm, gam_v)
        pltpu.sync_copy(bet_hbm, bet_v)

        lanes = lax.broadcasted_iota(jnp.int32, (L,), 0)
        zero = jnp.zeros((L,), jnp.float32)

        @plsc.parallel_loop(0, D, step=1, unroll=4)
        def _build_splats(j):
            cj = jnp.full((L,), j, jnp.int32)
            gspl[j] = plsc.load_gather(gam_v, [cj])
            bspl[j] = plsc.load_gather(bet_v, [cj])

        def fire_in(j, s):
            row0 = base + j * C
            pltpu.sync_copy(idx_hbm.at[pl.ds(row0, C)], idxs[s])
            pltpu.sync_copy(pid_hbm.at[pl.ds(row0, C)], pids[s])
            for k in range(SS):
                pltpu.async_copy(
                    item_hbm.at[idxs[s].at[pl.ds(k * SR, SR)]],
                    rows[s].at[pl.ds(k * SR, SR)],
                    sgs[s])

        def add_pos(s):
            for k in range(SS):
                pltpu.async_copy(
                    pos_v.at[pids[s].at[pl.ds(k * SR, SR)]],
                    rows[s].at[pl.ds(k * SR, SR)],
                    sgs[s], add=True)

        def wait_in(s, n=SS):
            for k in range(n):
                pltpu.make_async_copy(
                    item_hbm.at[idxs[s].at[pl.ds(0, SR)]],
                    rows[s].at[pl.ds(0, SR)],
                    sgs[s]).wait()

        def compute(s, o):
            rv, ob = rows[s], obs[o]

            def group_body(gi, _):
                g0 = gi * L
                il = gi // 8
                l0 = (gi % 8) * L
                ridx = g0 + lanes

                @plsc.parallel_loop(0, D, step=2, unroll=8,
                                    carry=(zero, zero, zero, zero))
                def col1(j, carry):
                    s0, q0, s1, q1 = carry
                    ca = jnp.full((L,), j, jnp.int32)
                    xa = plsc.load_gather(rv, [ridx, ca])
                    xT[j] = xa
                    xb = plsc.load_gather(rv, [ridx, ca + 1])
                    xT[j + 1] = xb
                    return s0 + xa, q0 + xa * xa, s1 + xb, q1 + xb * xb

                s0, q0, s1, q1 = col1
                mu = (s0 + s1) * (1.0 / D)
                var = (q0 + q1) * (1.0 / D) - mu * mu
                rstd = _rsqrt(var + 1e-5 / D)
                nmu = mu * rstd

                @plsc.parallel_loop(0, D, step=2, unroll=8)
                def col2(j):
                    ob[il, j, pl.ds(l0, L)] = xT[j] * rstd - nmu
                    ob[il, j + 1, pl.ds(l0, L)] = xT[j + 1] * rstd - nmu

                return 0

            lax.fori_loop(0, groups, group_body, 0)

            @plsc.parallel_loop(0, 2 * D, step=1, unroll=4)
            def affine(r):
                il = r // D
                j = r % D
                g = gspl[j]
                b = bspl[j]
                for t in range(128 // L):
                    v = ob[il, j, pl.ds(t * L, L)]
                    ob[il, j, pl.ds(t * L, L)] = v * g + b

        def fire_out(j, s):
            u = u0 + j // 4
            it = u // 32
            bt = u % 32
            is0 = (j % 4) * 2
            for il in range(2):
                i = it * 8 + is0 + il
                for tj in range(TJ):
                    pltpu.async_copy(
                        obs[s].at[il, pl.ds(tj * 8, 8), :],
                        out_hbm.at[i, tj, bt],
                        sos[s])

        def wait_out(s):
            for _ in range(2 * TJ):
                pltpu.make_async_copy(
                    obs[s].at[0, pl.ds(0, 8), :],
                    out_hbm.at[0, 0, 0],
                    sos[s]).wait()

        fire_in(0, 0)
        fire_in(1, 1)
        wait_in(0)
        add_pos(0)

        def hex_body(ci, _):
            for b in range(6):
                j = 6 * ci + b
                sj = b % 3
                s1 = (b + 1) % 3
                s2 = (b + 2) % 3
                o = b % 2

                @pl.when(j + 2 < nch)
                def _():
                    fire_in(j + 2, s2)

                @pl.when(j + 1 < nch)
                def _():
                    wait_in(s1)
                    add_pos(s1)

                wait_in(sj)

                @pl.when(j >= 2)
                def _():
                    wait_out(o)

                compute(sj, o)
                fire_out(j, o)
            return 0

        lax.fori_loop(0, nch // 6, hex_body, 0)
        for j in range(nch - nch % 6, nch):
            sj = j % 3
            s1 = (j + 1) % 3
            s2 = (j + 2) % 3
            o = j % 2
            if j + 2 < nch:
                fire_in(j + 2, s2)
            if j + 1 < nch:
                wait_in(s1)
                add_pos(s1)
            wait_in(sj)
            if j >= 2:
                wait_out(o)
            compute(sj, o)
            fire_out(j, o)
        wait_out(0)
        wait_out(1)

    return sc_kernel


def kernel(input_sequence, position_ids, item_table, pos_table, ln_gamma, ln_beta):
    I, B = input_sequence.shape
    V, D = item_table.shape
    P = pos_table.shape[0]
    N = I * B

    def native_flat(a):
        return (a.reshape(I // 8, 8, B // 128, 128)
                 .transpose(0, 2, 1, 3)
                 .reshape(N))

    sc = _make_sc_kernel(N, V, P, D)
    out5 = sc(
        native_flat(input_sequence),
        native_flat(position_ids),
        item_table,
        pos_table * (1.0 / math.sqrt(D)),
        ln_gamma,
        ln_beta,
    )
    return out5.transpose(0, 2, 4, 1, 3).reshape(I, B, D)


# speedup vs baseline: 3.2006x; 1.0311x over previous
import functools
import math

import jax
import jax.numpy as jnp
from jax import lax
from jax.experimental import pallas as pl
from jax.experimental.pallas import tpu as pltpu
from jax.experimental.pallas import tpu_sc as plsc

NC = 2
NS = 16
L = 16

C = 256
SS = 4
SR = C // SS


def _rsqrt(a):
    i = lax.bitcast_convert_type(a, jnp.int32)
    i = 0x5F3759DF - lax.shift_right_logical(i, 1)
    y = lax.bitcast_convert_type(i, jnp.float32)
    for _ in range(2):
        y = y * (1.5 - 0.5 * a * y * y)
    return y


def _make_sc_kernel(N, V, P, D):
    NW = NC * NS
    per_w = N // NW
    nch = per_w // C
    groups = C // L
    scale = math.sqrt(D)
    IT = N // (8 * 128 * 32)
    TJ = D // 8
    mesh = plsc.VectorSubcoreMesh(core_axis_name="c", subcore_axis_name="s")

    @functools.partial(
        pl.kernel,
        mesh=mesh,
        compiler_params=pltpu.CompilerParams(
            needs_layout_passes=False, use_tc_tiling_on_sc=False),
        out_type=jax.ShapeDtypeStruct((IT * 8, TJ, 32, 8, 128), jnp.float32),
        scratch_types=[
            pltpu.VMEM((C,), jnp.int32),
            pltpu.VMEM((C,), jnp.int32),
            pltpu.VMEM((C,), jnp.int32),
            pltpu.VMEM((C,), jnp.int32),
            pltpu.VMEM((C,), jnp.int32),
            pltpu.VMEM((C,), jnp.int32),
            pltpu.VMEM((C, D), jnp.float32),
            pltpu.VMEM((C, D), jnp.float32),
            pltpu.VMEM((C, D), jnp.float32),
            pltpu.VMEM_SHARED((200, 64), jnp.float32),
            pltpu.VMEM((2, D, 128), jnp.float32),
            pltpu.VMEM((2, D, 128), jnp.float32),
            pltpu.VMEM((D,), jnp.float32),
            pltpu.VMEM((D,), jnp.float32),
            pltpu.VMEM((2, 128), jnp.float32),
            pltpu.VMEM((2, 128), jnp.float32),
            pltpu.VMEM((D, L), jnp.float32),
            pltpu.VMEM((D, L), jnp.float32),
            pltpu.SemaphoreType.DMA,
            pltpu.SemaphoreType.DMA,
            pltpu.SemaphoreType.DMA,
            pltpu.SemaphoreType.DMA,
            pltpu.SemaphoreType.DMA,
        ],
    )
    def sc_kernel(idx_hbm, pid_hbm, item_hbm, pos_hbm, gam_hbm, bet_hbm,
                  out_hbm, idx0, idx1, idx2, pid0, pid1, pid2, rows0, rows1, rows2,
                  pos_v, ob0, ob1, gam_v, bet_v, rstds, nmus, gspl, bspl,
                  sg0, sg1, sg2, so0, so1):
        idxs, pids, rows = [idx0, idx1, idx2], [pid0, pid1, pid2], [rows0, rows1, rows2]
        obs = [ob0, ob1]
        sgs, sos = [sg0, sg1, sg2], [so0, so1]
        wid = lax.axis_index("s") * NC + lax.axis_index("c")
        base = wid * per_w
        u0 = base // 1024

        @pl.when(lax.axis_index("s") == 0)
        def _():
            pltpu.sync_copy(pos_hbm, pos_v)
        plsc.subcore_barrier()
        pltpu.sync_copy(gam_hbm, gam_v)
        pltpu.sync_copy(bet_hbm, bet_v)

        lanes = lax.broadcasted_iota(jnp.int32, (L,), 0)
        zero = jnp.zeros((L,), jnp.float32)

        @plsc.parallel_loop(0, D, step=1, unroll=4)
        def _build_splats(j):
            cj = jnp.full((L,), j, jnp.int32)
            gspl[j] = plsc.load_gather(gam_v, [cj])
            bspl[j] = plsc.load_gather(bet_v, [cj])

        def fire_in(j, s):
            row0 = base + j * C
            pltpu.sync_copy(idx_hbm.at[pl.ds(row0, C)], idxs[s])
            pltpu.sync_copy(pid_hbm.at[pl.ds(row0, C)], pids[s])
            for k in range(SS):
                pltpu.async_copy(
                    item_hbm.at[idxs[s].at[pl.ds(k * SR, SR)]],
                    rows[s].at[pl.ds(k * SR, SR)],
                    sgs[s])

        def add_pos(s):
            for k in range(SS):
                pltpu.async_copy(
                    pos_v.at[pids[s].at[pl.ds(k * SR, SR)]],
                    rows[s].at[pl.ds(k * SR, SR)],
                    sgs[s], add=True)

        def wait_in(s, n=SS):
            for k in range(n):
                pltpu.make_async_copy(
                    item_hbm.at[idxs[s].at[pl.ds(0, SR)]],
                    rows[s].at[pl.ds(0, SR)],
                    sgs[s]).wait()

        def compute(s, o):
            rv, ob = rows[s], obs[o]

            def group_body(gi, _):
                g0 = gi * L
                il = gi // 8
                l0 = (gi % 8) * L
                ridx = g0 + lanes

                @plsc.parallel_loop(0, D, step=2, unroll=8,
                                    carry=(zero, zero, zero, zero))
                def col1(j, carry):
                    s0, q0, s1, q1 = carry
                    ca = jnp.full((L,), j, jnp.int32)
                    xa = plsc.load_gather(rv, [ridx, ca])
                    ob[il, j, pl.ds(l0, L)] = xa
                    xb = plsc.load_gather(rv, [ridx, ca + 1])
                    ob[il, j + 1, pl.ds(l0, L)] = xb
                    return s0 + xa, q0 + xa * xa, s1 + xb, q1 + xb * xb

                s0, q0, s1, q1 = col1
                mu = (s0 + s1) * (1.0 / D)
                var = (q0 + q1) * (1.0 / D) - mu * mu
                rstd = _rsqrt(var + 1e-5 / D)
                rstds[il, pl.ds(l0, L)] = rstd
                nmus[il, pl.ds(l0, L)] = mu * rstd
                return 0

            lax.fori_loop(0, groups, group_body, 0)

            for il in range(2):
                rs = [rstds[il, pl.ds(t * L, L)] for t in range(128 // L)]
                ns = [nmus[il, pl.ds(t * L, L)] for t in range(128 // L)]

                @plsc.parallel_loop(0, D, step=1, unroll=2)
                def norm_affine(j):
                    g = gspl[j]
                    b = bspl[j]
                    for t in range(128 // L):
                        v = ob[il, j, pl.ds(t * L, L)]
                        ob[il, j, pl.ds(t * L, L)] = (v * rs[t] - ns[t]) * g + b

        def fire_out(j, s):
            u = u0 + j // 4
            it = u // 32
            bt = u % 32
            is0 = (j % 4) * 2
            for il in range(2):
                i = it * 8 + is0 + il
                for tj in range(TJ):
                    pltpu.async_copy(
                        obs[s].at[il, pl.ds(tj * 8, 8), :],
                        out_hbm.at[i, tj, bt],
                        sos[s])

        def wait_out(s):
            for _ in range(2 * TJ):
                pltpu.make_async_copy(
                    obs[s].at[0, pl.ds(0, 8), :],
                    out_hbm.at[0, 0, 0],
                    sos[s]).wait()

        fire_in(0, 0)
        fire_in(1, 1)
        wait_in(0)
        add_pos(0)

        def hex_body(ci, _):
            for b in range(6):
                j = 6 * ci + b
                sj = b % 3
                s1 = (b + 1) % 3
                s2 = (b + 2) % 3
                o = b % 2

                @pl.when(j + 2 < nch)
                def _():
                    fire_in(j + 2, s2)

                @pl.when(j + 1 < nch)
                def _():
                    wait_in(s1)
                    add_pos(s1)

                wait_in(sj)

                @pl.when(j >= 2)
                def _():
                    wait_out(o)

                compute(sj, o)
                fire_out(j, o)
            return 0

        lax.fori_loop(0, nch // 6, hex_body, 0)
        for j in range(nch - nch % 6, nch):
            sj = j % 3
            s1 = (j + 1) % 3
            s2 = (j + 2) % 3
            o = j % 2
            if j + 2 < nch:
                fire_in(j + 2, s2)
            if j + 1 < nch:
                wait_in(s1)
                add_pos(s1)
            wait_in(sj)
            if j >= 2:
                wait_out(o)
            compute(sj, o)
            fire_out(j, o)
        wait_out(0)
        wait_out(1)

    return sc_kernel


def kernel(input_sequence, position_ids, item_table, pos_table, ln_gamma, ln_beta):
    I, B = input_sequence.shape
    V, D = item_table.shape
    P = pos_table.shape[0]
    N = I * B

    def native_flat(a):
        return (a.reshape(I // 8, 8, B // 128, 128)
                 .transpose(0, 2, 1, 3)
                 .reshape(N))

    sc = _make_sc_kernel(N, V, P, D)
    out5 = sc(
        native_flat(input_sequence),
        native_flat(position_ids),
        item_table,
        pos_table * (1.0 / math.sqrt(D)),
        ln_gamma,
        ln_beta,
    )
    return out5.transpose(0, 2, 4, 1, 3).reshape(I, B, D)
